# baseline (speedup 1.0000x reference)
import functools

import jax
import jax.numpy as jnp
from jax import lax
from jax.experimental import pallas as pl
from jax.experimental.pallas import tpu as pltpu
from jax.experimental.pallas import tpu_sc as plsc

B = 16384
EMB_DIM = 32
BERT_DIM = 768
L = 16
NC, NS = 2, 16
NW = NC * NS
BPW = B // NW
CH = 64
NCHUNK = BPW // CH
NGROUP = CH // L

_mesh = plsc.VectorSubcoreMesh(core_axis_name="c", subcore_axis_name="s")

_GATHER_DNUMS = lax.GatherDimensionNumbers(
    offset_dims=(), collapsed_slice_dims=(0,), start_index_map=(0,))


def _lane_shuffle(v, idx):
    return lax.gather(v, idx[:, None], _GATHER_DNUMS, (1,),
                      mode=lax.GatherScatterMode.PROMISE_IN_BOUNDS)


@functools.partial(
    pl.kernel,
    out_type=jax.ShapeDtypeStruct((B,), jnp.float32),
    mesh=_mesh,
    compiler_params=pltpu.CompilerParams(use_tc_tiling_on_sc=False),
    scratch_types=[
        pltpu.VMEM((NCHUNK, CH), jnp.int32),
        pltpu.VMEM((NCHUNK, CH), jnp.int32),
        pltpu.VMEM((CH, BERT_DIM), jnp.float32),
        pltpu.VMEM((CH, BERT_DIM), jnp.float32),
        pltpu.VMEM((CH, EMB_DIM), jnp.float32),
        pltpu.VMEM((CH, EMB_DIM), jnp.float32),
        pltpu.VMEM((CH,), jnp.float32),
        pltpu.VMEM((CH,), jnp.float32),
        pltpu.VMEM((BPW,), jnp.float32),
        pltpu.VMEM((L,), jnp.float32),
        pltpu.SemaphoreType.DMA,
    ],
)
def _sc_kernel(uid2_hbm, iid2_hbm, uemb_hbm, iemb_hbm, utext_hbm, itext_hbm,
               ubias_hbm, ibias_hbm, bias16_hbm, out_hbm,
               uid_v, iid_v, ut_v, it_v, ue_v, ie_v, ub_v, ib_v, out_v,
               bias_v, sem):
    wid = lax.axis_index("s") * NC + lax.axis_index("c")
    row0 = wid * NCHUNK

    pltpu.sync_copy(uid2_hbm.at[pl.ds(row0, NCHUNK)], uid_v)
    pltpu.sync_copy(iid2_hbm.at[pl.ds(row0, NCHUNK)], iid_v)
    pltpu.sync_copy(bias16_hbm, bias_v)
    bias_vec = bias_v[pl.ds(0, L)]
    lane = lax.iota(jnp.int32, L)

    def chunk_body(j, carry):
        uids = uid_v.at[j]
        iids = iid_v.at[j]
        cps = [
            pltpu.make_async_copy(utext_hbm.at[uids], ut_v, sem),
            pltpu.make_async_copy(itext_hbm.at[iids], it_v, sem),
            pltpu.make_async_copy(uemb_hbm.at[uids], ue_v, sem),
            pltpu.make_async_copy(iemb_hbm.at[iids], ie_v, sem),
            pltpu.make_async_copy(ubias_hbm.at[uids], ub_v, sem),
            pltpu.make_async_copy(ibias_hbm.at[iids], ib_v, sem),
        ]
        for c in cps:
            c.start()
        for c in cps:
            c.wait()

        def group_body(g, carry2):
            r = bias_vec
            for p in range(L):
                i = g * L + p
                acc = ue_v[i, pl.ds(0, L)] * ie_v[i, pl.ds(0, L)]
                acc = acc + ue_v[i, pl.ds(L, L)] * ie_v[i, pl.ds(L, L)]
                for t in range(BERT_DIM // L):
                    acc = acc + (ut_v[i, pl.ds(t * L, L)]
                                 * it_v[i, pl.ds(t * L, L)])
                for sh in (8, 4, 2, 1):
                    acc = acc + _lane_shuffle(acc, lane ^ sh)
                r = jnp.where(lane == p, acc, r)
            goff = g * L
            r = r + ub_v[pl.ds(goff, L)] + ib_v[pl.ds(goff, L)]
            out_v[pl.ds(j * CH + goff, L)] = r
            return carry2

        lax.fori_loop(0, NGROUP, group_body, 0)
        return carry

    lax.fori_loop(0, NCHUNK, chunk_body, 0)
    pltpu.sync_copy(out_v, out_hbm.at[pl.ds(wid * BPW, BPW)])


def kernel(user_ids, item_ids, user_emb_w, item_emb_w, user_text_w,
           item_text_w, user_bias, item_bias, bias):
    uid2 = user_ids.reshape(B // CH, CH)
    iid2 = item_ids.reshape(B // CH, CH)
    bias16 = jnp.broadcast_to(bias, (L,))
    out = _sc_kernel(uid2, iid2, user_emb_w, item_emb_w, user_text_w,
                     item_text_w, user_bias, item_bias, bias16)
    return out[:, None]


# speedup vs baseline: 1.2727x; 1.2727x over previous
import functools

import jax
import jax.numpy as jnp
from jax import lax
from jax.experimental import pallas as pl
from jax.experimental.pallas import tpu as pltpu
from jax.experimental.pallas import tpu_sc as plsc

B = 16384
EMB_DIM = 32
BERT_DIM = 768
L = 16
NC, NS = 2, 16
NW = NC * NS
BPW = B // NW
CH = 32
NCHUNK = BPW // CH

_mesh = plsc.VectorSubcoreMesh(core_axis_name="c", subcore_axis_name="s")

_GATHER_DNUMS = lax.GatherDimensionNumbers(
    offset_dims=(), collapsed_slice_dims=(0,), start_index_map=(0,))


def _lane_shuffle(v, idx):
    return lax.gather(v, idx[:, None], _GATHER_DNUMS, (1,),
                      mode=lax.GatherScatterMode.PROMISE_IN_BOUNDS)


@functools.partial(
    pl.kernel,
    out_type=jax.ShapeDtypeStruct((B,), jnp.float32),
    mesh=_mesh,
    compiler_params=pltpu.CompilerParams(use_tc_tiling_on_sc=False,
                                         needs_layout_passes=False),
    scratch_types=[
        pltpu.VMEM((BPW,), jnp.int32),
        pltpu.VMEM((BPW,), jnp.int32),
        pltpu.VMEM((2, CH, BERT_DIM), jnp.float32),
        pltpu.VMEM((2, CH, BERT_DIM), jnp.float32),
        pltpu.VMEM((2, CH, EMB_DIM), jnp.float32),
        pltpu.VMEM((2, CH, EMB_DIM), jnp.float32),
        pltpu.VMEM((2, CH), jnp.float32),
        pltpu.VMEM((2, CH), jnp.float32),
        pltpu.VMEM((BPW,), jnp.float32),
        pltpu.VMEM((L,), jnp.float32),
        pltpu.SemaphoreType.DMA((2,)),
    ],
)
def _sc_kernel(uid_hbm, iid_hbm, uemb_hbm, iemb_hbm, utext_hbm, itext_hbm,
               ubias_hbm, ibias_hbm, bias16_hbm, out_hbm,
               uid_v, iid_v, ut_v, it_v, ue_v, ie_v, ub_v, ib_v, out_v,
               bias_v, sem):
    wid = lax.axis_index("s") * NC + lax.axis_index("c")
    base = wid * BPW

    pltpu.sync_copy(uid_hbm.at[pl.ds(base, BPW)], uid_v)
    pltpu.sync_copy(iid_hbm.at[pl.ds(base, BPW)], iid_v)
    pltpu.sync_copy(bias16_hbm, bias_v)
    bias_vec = bias_v[pl.ds(0, L)]
    lane = lax.iota(jnp.int32, L)
    lane0 = lane == 0

    def chunk_copies(j, p):
        uids = uid_v.at[pl.ds(j * CH, CH)]
        iids = iid_v.at[pl.ds(j * CH, CH)]
        return [
            pltpu.make_async_copy(utext_hbm.at[uids], ut_v.at[p], sem.at[p]),
            pltpu.make_async_copy(itext_hbm.at[iids], it_v.at[p], sem.at[p]),
            pltpu.make_async_copy(uemb_hbm.at[uids], ue_v.at[p], sem.at[p]),
            pltpu.make_async_copy(iemb_hbm.at[iids], ie_v.at[p], sem.at[p]),
            pltpu.make_async_copy(ubias_hbm.at[uids], ub_v.at[p], sem.at[p]),
            pltpu.make_async_copy(ibias_hbm.at[iids], ib_v.at[p], sem.at[p]),
        ]

    for c in chunk_copies(0, 0):
        c.start()

    def chunk_body(j, carry):
        p = lax.rem(j, 2)
        q = 1 - p

        @pl.when(j < NCHUNK - 1)
        def _issue_next():
            for c in chunk_copies(j + 1, q):
                c.start()

        for c in chunk_copies(j, p):
            c.wait()

        def pair_body(i, carry2):
            acc = ue_v[p, i, pl.ds(0, L)] * ie_v[p, i, pl.ds(0, L)]
            acc = acc + ue_v[p, i, pl.ds(L, L)] * ie_v[p, i, pl.ds(L, L)]
            for t in range(BERT_DIM // L):
                acc = acc + (ut_v[p, i, pl.ds(t * L, L)]
                             * it_v[p, i, pl.ds(t * L, L)])
            for sh in (8, 4, 2, 1):
                acc = acc + _lane_shuffle(acc, lane ^ sh)
            pos = jnp.broadcast_to(j * CH + i, (L,)).astype(jnp.int32)
            plsc.store_scatter(out_v, [pos], acc, mask=lane0)
            return carry2

        lax.fori_loop(0, CH, pair_body, 0)

        for gg in range(CH // L):
            off = j * CH + gg * L
            r = (out_v[pl.ds(off, L)] + ub_v[p, pl.ds(gg * L, L)]
                 + ib_v[p, pl.ds(gg * L, L)] + bias_vec)
            out_v[pl.ds(off, L)] = r
        return carry

    lax.fori_loop(0, NCHUNK, chunk_body, 0)
    pltpu.sync_copy(out_v, out_hbm.at[pl.ds(base, BPW)])


def kernel(user_ids, item_ids, user_emb_w, item_emb_w, user_text_w,
           item_text_w, user_bias, item_bias, bias):
    bias16 = jnp.broadcast_to(bias, (L,))
    out = _sc_kernel(user_ids, item_ids, user_emb_w, item_emb_w, user_text_w,
                     item_text_w, user_bias, item_bias, bias16)
    return out[:, None]


# speedup vs baseline: 5.7755x; 4.5380x over previous
import functools

import jax
import jax.numpy as jnp
from jax import lax
from jax.experimental import pallas as pl
from jax.experimental.pallas import tpu as pltpu
from jax.experimental.pallas import tpu_sc as plsc

B = 16384
EMB_DIM = 32
BERT_DIM = 768
L = 16
NC, NS = 2, 16
NW = NC * NS
BPW = B // NW
CH = 32
NCHUNK = BPW // CH
EPR = 128 // EMB_DIM

_mesh = plsc.VectorSubcoreMesh(core_axis_name="c", subcore_axis_name="s")

_GATHER_DNUMS = lax.GatherDimensionNumbers(
    offset_dims=(), collapsed_slice_dims=(0,), start_index_map=(0,))


def _lane_shuffle(v, idx):
    return lax.gather(v, idx[:, None], _GATHER_DNUMS, (1,),
                      mode=lax.GatherScatterMode.PROMISE_IN_BOUNDS)


@functools.partial(
    pl.kernel,
    out_type=jax.ShapeDtypeStruct((B,), jnp.float32),
    mesh=_mesh,
    compiler_params=pltpu.CompilerParams(needs_layout_passes=False),
    scratch_types=[
        pltpu.VMEM((BPW,), jnp.int32),
        pltpu.VMEM((BPW,), jnp.int32),
        pltpu.VMEM((BPW,), jnp.int32),
        pltpu.VMEM((BPW,), jnp.int32),
        pltpu.VMEM((2, CH, BERT_DIM), jnp.float32),
        pltpu.VMEM((2, CH, BERT_DIM), jnp.float32),
        pltpu.VMEM((2, CH, 128), jnp.float32),
        pltpu.VMEM((2, CH, 128), jnp.float32),
        pltpu.VMEM((2, CH), jnp.float32),
        pltpu.VMEM((2, CH), jnp.float32),
        pltpu.VMEM((BPW,), jnp.float32),
        pltpu.VMEM((L,), jnp.float32),
        pltpu.SemaphoreType.DMA((2,)),
    ],
)
def _sc_kernel(uid_hbm, iid_hbm, uemb_hbm, iemb_hbm, utext_hbm, itext_hbm,
               ubias_hbm, ibias_hbm, bias16_hbm, out_hbm,
               uid_v, iid_v, urow_v, irow_v, ut_v, it_v, ue_v, ie_v,
               ub_v, ib_v, out_v, bias_v, sem):
    wid = lax.axis_index("s") * NC + lax.axis_index("c")
    base = wid * BPW

    pltpu.sync_copy(uid_hbm.at[pl.ds(base, BPW)], uid_v)
    pltpu.sync_copy(iid_hbm.at[pl.ds(base, BPW)], iid_v)
    pltpu.sync_copy(bias16_hbm, bias_v)
    bias_vec = bias_v[pl.ds(0, L)]
    lane = lax.iota(jnp.int32, L)
    lane0 = lane == 0

    def row_idx_body(g, carry):
        off = g * L
        uv = uid_v[pl.ds(off, L)]
        iv = iid_v[pl.ds(off, L)]
        urow_v[pl.ds(off, L)] = lax.shift_right_logical(uv, 2)
        irow_v[pl.ds(off, L)] = lax.shift_right_logical(iv, 2)
        return carry

    lax.fori_loop(0, BPW // L, row_idx_body, 0)

    def chunk_copies(j, p):
        uids = uid_v.at[pl.ds(j * CH, CH)]
        iids = iid_v.at[pl.ds(j * CH, CH)]
        urows = urow_v.at[pl.ds(j * CH, CH)]
        irows = irow_v.at[pl.ds(j * CH, CH)]
        return [
            pltpu.make_async_copy(utext_hbm.at[uids], ut_v.at[p], sem.at[p]),
            pltpu.make_async_copy(itext_hbm.at[iids], it_v.at[p], sem.at[p]),
            pltpu.make_async_copy(uemb_hbm.at[urows], ue_v.at[p], sem.at[p]),
            pltpu.make_async_copy(iemb_hbm.at[irows], ie_v.at[p], sem.at[p]),
            pltpu.make_async_copy(ubias_hbm.at[uids], ub_v.at[p], sem.at[p]),
            pltpu.make_async_copy(ibias_hbm.at[iids], ib_v.at[p], sem.at[p]),
        ]

    for c in chunk_copies(0, 0):
        c.start()

    def chunk_body(j, carry):
        p = lax.rem(j, 2)
        q = 1 - p

        @pl.when(j < NCHUNK - 1)
        def _issue_next():
            for c in chunk_copies(j + 1, q):
                c.start()

        for c in chunk_copies(j, p):
            c.wait()

        def pair_body(i, carry2):
            pig = lax.rem(i, L)
            grp = j * CH + i - pig
            pos_in_grp = jnp.broadcast_to(pig, (L,))
            uid_b = _lane_shuffle(uid_v[pl.ds(grp, L)], pos_in_grp)
            iid_b = _lane_shuffle(iid_v[pl.ds(grp, L)], pos_in_grp)
            uq = (uid_b & (EPR - 1)) * EMB_DIM + lane
            iq = (iid_b & (EPR - 1)) * EMB_DIM + lane
            i_b = jnp.broadcast_to(i, (L,)).astype(jnp.int32)
            p_b = jnp.broadcast_to(p, (L,)).astype(jnp.int32)
            ue0 = plsc.load_gather(ue_v, [p_b, i_b, uq])
            ie0 = plsc.load_gather(ie_v, [p_b, i_b, iq])
            ue1 = plsc.load_gather(ue_v, [p_b, i_b, uq + L])
            ie1 = plsc.load_gather(ie_v, [p_b, i_b, iq + L])
            acc = ue0 * ie0 + ue1 * ie1
            for t in range(BERT_DIM // L):
                acc = acc + (ut_v[p, i, pl.ds(t * L, L)]
                             * it_v[p, i, pl.ds(t * L, L)])
            for sh in (8, 4, 2, 1):
                acc = acc + _lane_shuffle(acc, lane ^ sh)
            pos = jnp.broadcast_to(j * CH + i, (L,)).astype(jnp.int32)
            plsc.store_scatter(out_v, [pos], acc, mask=lane0)
            return carry2

        lax.fori_loop(0, CH, pair_body, 0)

        for gg in range(CH // L):
            off = j * CH + gg * L
            r = (out_v[pl.ds(off, L)] + ub_v[p, pl.ds(gg * L, L)]
                 + ib_v[p, pl.ds(gg * L, L)] + bias_vec)
            out_v[pl.ds(off, L)] = r
        return carry

    lax.fori_loop(0, NCHUNK, chunk_body, 0)
    pltpu.sync_copy(out_v, out_hbm.at[pl.ds(base, BPW)])


def kernel(user_ids, item_ids, user_emb_w, item_emb_w, user_text_w,
           item_text_w, user_bias, item_bias, bias):
    uemb2 = user_emb_w.reshape(N_EMB_ROWS, 128)
    iemb2 = item_emb_w.reshape(N_EMB_ROWS, 128)
    bias16 = jnp.broadcast_to(bias, (L,))
    out = _sc_kernel(user_ids, item_ids, uemb2, iemb2, user_text_w,
                     item_text_w, user_bias, item_bias, bias16)
    return out[:, None]


N_EMB_ROWS = 100000 * EMB_DIM // 128
